# Initial kernel scaffold; baseline (speedup 1.0000x reference)
import jax
import jax.numpy as jnp
from jax.experimental import pallas as pl


def kernel(inputs, weight):
    raise NotImplementedError("write your pallas kernel here")


# speedup vs baseline: 3.3266x; 3.3266x over previous
import jax
import jax.numpy as jnp
from jax.experimental import pallas as pl
from jax.experimental.pallas import tpu as pltpu
from jax.experimental.pallas import tpu_sc as plsc

_N = 8192
_D = 32
_BM = 256
_GW = 128
_COMMIT = 0.25


def _dist_argmin_kernel(x_ref, wt_ref, sx_ref, sw_ref, idx_ref, part_ref):
    x = x_ref[...]
    wt = wt_ref[...]
    mm = jax.lax.dot_general(
        x, wt, (((1,), (0,)), ((), ())),
        preferred_element_type=jnp.float32,
        precision=jax.lax.Precision.DEFAULT)
    dist = (sx_ref[...] + sw_ref[...]) - 2.0 * mm
    bmin = jnp.min(dist, axis=1, keepdims=True)
    iota = jax.lax.broadcasted_iota(jnp.int32, (_BM, _N), 1)
    idx = jnp.min(jnp.where(dist == bmin, iota, jnp.int32(_N)), axis=1)
    idx_ref[0, 0, :] = idx
    part_ref[0, 0, :] = jnp.broadcast_to(jnp.sum(bmin), (128,))


def _argmin_and_losssum(flat, weight_t, sx, sw):
    grid = flat.shape[0] // _BM
    idx3, part = pl.pallas_call(
        _dist_argmin_kernel,
        grid=(grid,),
        in_specs=[
            pl.BlockSpec((_BM, _D), lambda i: (i, 0)),
            pl.BlockSpec((_D, _N), lambda i: (0, 0)),
            pl.BlockSpec((_BM, 1), lambda i: (i, 0)),
            pl.BlockSpec((1, _N), lambda i: (0, 0)),
        ],
        out_specs=[
            pl.BlockSpec((1, 1, _BM), lambda i: (i, 0, 0)),
            pl.BlockSpec((1, 1, 128), lambda i: (i, 0, 0)),
        ],
        out_shape=[
            jax.ShapeDtypeStruct((grid, 1, _BM), jnp.int32),
            jax.ShapeDtypeStruct((grid, 1, 128), jnp.float32),
        ],
    )(flat, weight_t, sx, sw)
    return idx3, part


def _sc_gather(weight_pad, idx_row):
    n_idx = idx_row.shape[1]
    lanes = weight_pad.shape[1]
    mesh = plsc.VectorSubcoreMesh(
        core_axis_name="core", subcore_axis_name="subcore")

    @pl.kernel(
        out_type=jax.ShapeDtypeStruct((n_idx, lanes), weight_pad.dtype),
        mesh=mesh)
    def gather_kernel(w_hbm, i_hbm, o_hbm):
        def body(i_vmem, o_vmem):
            pltpu.sync_copy(w_hbm.at[i_vmem.at[0]], o_vmem)

        pltpu.emit_pipeline(
            body,
            grid=(n_idx // _GW,),
            in_specs=[pl.BlockSpec((1, _GW), index_map=lambda i: (0, i))],
            out_specs=[pl.BlockSpec((_GW, lanes), index_map=lambda i: (i, 0))],
            core_axis_name="subcore",
            dimension_semantics=(pltpu.PARALLEL,),
        )(i_hbm, o_hbm)

    return gather_kernel(weight_pad, idx_row)


def kernel(inputs, weight):
    b, nc, h, w = inputs.shape
    x = jnp.transpose(inputs, (0, 2, 3, 1))
    flat = x.reshape(b * h * w, nc)
    sx = jnp.sum(flat ** 2, axis=1, keepdims=True)
    sw = jnp.sum(weight ** 2, axis=1)[None, :]
    idx3, part = _argmin_and_losssum(flat, weight.T, sx, sw)
    idx_row = idx3.reshape(1, flat.shape[0])
    weight_pad = jnp.pad(weight, ((0, 0), (0, 128 - _D)))
    quant_flat = _sc_gather(weight_pad, idx_row)[:, :_D]
    quant_flat = flat + (quant_flat - flat)
    total = jnp.sum(part[:, 0, 0])
    loss = (1.0 + _COMMIT) * total / (flat.shape[0] * nc)
    quantised = quant_flat.reshape(x.shape)
    return loss, jnp.transpose(quantised, (0, 3, 1, 2))


# speedup vs baseline: 3.6795x; 1.1061x over previous
import jax
import jax.numpy as jnp
from jax.experimental import pallas as pl
from jax.experimental.pallas import tpu as pltpu
from jax.experimental.pallas import tpu_sc as plsc

_N = 8192
_D = 32
_BM = 256
_CW = 512
_GW = 128
_COMMIT = 0.25


def _dist_argmin_kernel(x_ref, wt2_ref, sx_ref, sw_ref, idx_ref, part_ref):
    x = x_ref[...]
    mm2 = jax.lax.dot_general(
        x, wt2_ref[...], (((1,), (0,)), ((), ())),
        preferred_element_type=jnp.float32,
        precision=jax.lax.Precision.DEFAULT)
    sx = sx_ref[...]
    m = jnp.full((_BM, _CW), jnp.inf, jnp.float32)
    cidx = jnp.zeros((_BM, _CW), jnp.float32)
    for c in range(_N // _CW):
        sl = slice(c * _CW, (c + 1) * _CW)
        d = (sx + sw_ref[:, sl]) - mm2[:, sl]
        better = d < m
        cidx = jnp.where(better, jnp.float32(c), cidx)
        m = jnp.minimum(m, d)
    bmin = jnp.min(m, axis=1, keepdims=True)
    lane = jax.lax.broadcasted_iota(jnp.int32, (_BM, _CW), 1).astype(jnp.float32)
    jf = jnp.where(m == bmin, cidx * _CW + lane, jnp.float32(_N))
    idx = jnp.min(jf, axis=1).astype(jnp.int32)
    idx_ref[0, 0, :] = idx
    part_ref[0, 0, :] = jnp.broadcast_to(jnp.sum(bmin), (128,))


def _argmin_and_losssum(flat, weight_t, sx, sw):
    grid = flat.shape[0] // _BM
    idx3, part = pl.pallas_call(
        _dist_argmin_kernel,
        grid=(grid,),
        in_specs=[
            pl.BlockSpec((_BM, _D), lambda i: (i, 0)),
            pl.BlockSpec((_D, _N), lambda i: (0, 0)),
            pl.BlockSpec((_BM, 1), lambda i: (i, 0)),
            pl.BlockSpec((1, _N), lambda i: (0, 0)),
        ],
        out_specs=[
            pl.BlockSpec((1, 1, _BM), lambda i: (i, 0, 0)),
            pl.BlockSpec((1, 1, 128), lambda i: (i, 0, 0)),
        ],
        out_shape=[
            jax.ShapeDtypeStruct((grid, 1, _BM), jnp.int32),
            jax.ShapeDtypeStruct((grid, 1, 128), jnp.float32),
        ],
    )(flat, weight_t, sx, sw)
    return idx3, part


def _sc_gather(weight_pad, idx_row):
    n_idx = idx_row.shape[1]
    lanes = weight_pad.shape[1]
    mesh = plsc.VectorSubcoreMesh(
        core_axis_name="core", subcore_axis_name="subcore")

    @pl.kernel(
        out_type=jax.ShapeDtypeStruct((n_idx, lanes), weight_pad.dtype),
        mesh=mesh)
    def gather_kernel(w_hbm, i_hbm, o_hbm):
        def body(i_vmem, o_vmem):
            pltpu.sync_copy(w_hbm.at[i_vmem.at[0]], o_vmem)

        pltpu.emit_pipeline(
            body,
            grid=(n_idx // _GW,),
            in_specs=[pl.BlockSpec((1, _GW), index_map=lambda i: (0, i))],
            out_specs=[pl.BlockSpec((_GW, lanes), index_map=lambda i: (i, 0))],
            core_axis_name="subcore",
            dimension_semantics=(pltpu.PARALLEL,),
        )(i_hbm, o_hbm)

    return gather_kernel(weight_pad, idx_row)


def kernel(inputs, weight):
    b, nc, h, w = inputs.shape
    x = jnp.transpose(inputs, (0, 2, 3, 1))
    flat = x.reshape(b * h * w, nc)
    sx = jnp.sum(flat ** 2, axis=1, keepdims=True)
    sw = jnp.sum(weight ** 2, axis=1)[None, :]
    idx3, part = _argmin_and_losssum(flat, (2.0 * weight).T, sx, sw)
    idx_row = idx3.reshape(1, flat.shape[0])
    weight_pad = jnp.pad(weight, ((0, 0), (0, 128 - _D)))
    quant_flat = _sc_gather(weight_pad, idx_row)[:, :_D]
    quant_flat = flat + (quant_flat - flat)
    total = jnp.sum(part[:, 0, 0])
    loss = (1.0 + _COMMIT) * total / (flat.shape[0] * nc)
    quantised = quant_flat.reshape(x.shape)
    return loss, jnp.transpose(quantised, (0, 3, 1, 2))


# speedup vs baseline: 3.8761x; 1.0534x over previous
import jax
import jax.numpy as jnp
from jax.experimental import pallas as pl
from jax.experimental.pallas import tpu as pltpu
from jax.experimental.pallas import tpu_sc as plsc

_N = 8192
_D = 32
_BM = 256
_CW = 256
_RG = 32
_GW = 128
_COMMIT = 0.25


def _dist_argmin_kernel(x_ref, wt2_ref, sx_ref, sw_ref, idx_ref, part_ref):
    x = x_ref[...]
    mm2 = jax.lax.dot_general(
        x, wt2_ref[...], (((1,), (0,)), ((), ())),
        preferred_element_type=jnp.float32,
        precision=jax.lax.Precision.DEFAULT)
    lane = jax.lax.broadcasted_iota(jnp.int32, (_RG, _CW), 1).astype(jnp.float32)
    lsum = jnp.float32(0.0)
    for r in range(_BM // _RG):
        rs = slice(r * _RG, (r + 1) * _RG)
        sx = sx_ref[rs, :]
        m = jnp.full((_RG, _CW), jnp.inf, jnp.float32)
        cidx = jnp.zeros((_RG, _CW), jnp.float32)
        for c in range(_N // _CW):
            cs = slice(c * _CW, (c + 1) * _CW)
            d = (sx + sw_ref[:, cs]) - mm2[rs, cs]
            better = d < m
            cidx = jnp.where(better, jnp.float32(c), cidx)
            m = jnp.minimum(m, d)
        bmin = jnp.min(m, axis=1, keepdims=True)
        jf = jnp.where(m == bmin, cidx * _CW + lane, jnp.float32(_N))
        idx_ref[0, 0, rs] = jnp.min(jf, axis=1).astype(jnp.int32)
        lsum = lsum + jnp.sum(bmin)
    part_ref[0, 0, :] = jnp.broadcast_to(lsum, (128,))


def _argmin_and_losssum(flat, weight_t, sx, sw):
    grid = flat.shape[0] // _BM
    idx3, part = pl.pallas_call(
        _dist_argmin_kernel,
        grid=(grid,),
        in_specs=[
            pl.BlockSpec((_BM, _D), lambda i: (i, 0)),
            pl.BlockSpec((_D, _N), lambda i: (0, 0)),
            pl.BlockSpec((_BM, 1), lambda i: (i, 0)),
            pl.BlockSpec((1, _N), lambda i: (0, 0)),
        ],
        out_specs=[
            pl.BlockSpec((1, 1, _BM), lambda i: (i, 0, 0)),
            pl.BlockSpec((1, 1, 128), lambda i: (i, 0, 0)),
        ],
        out_shape=[
            jax.ShapeDtypeStruct((grid, 1, _BM), jnp.int32),
            jax.ShapeDtypeStruct((grid, 1, 128), jnp.float32),
        ],
    )(flat, weight_t, sx, sw)
    return idx3, part


def _sc_gather(weight_pad, idx_row):
    n_idx = idx_row.shape[1]
    lanes = weight_pad.shape[1]
    mesh = plsc.VectorSubcoreMesh(
        core_axis_name="core", subcore_axis_name="subcore")

    @pl.kernel(
        out_type=jax.ShapeDtypeStruct((n_idx, lanes), weight_pad.dtype),
        mesh=mesh)
    def gather_kernel(w_hbm, i_hbm, o_hbm):
        def body(i_vmem, o_vmem):
            pltpu.sync_copy(w_hbm.at[i_vmem.at[0]], o_vmem)

        pltpu.emit_pipeline(
            body,
            grid=(n_idx // _GW,),
            in_specs=[pl.BlockSpec((1, _GW), index_map=lambda i: (0, i))],
            out_specs=[pl.BlockSpec((_GW, lanes), index_map=lambda i: (i, 0))],
            core_axis_name="subcore",
            dimension_semantics=(pltpu.PARALLEL,),
        )(i_hbm, o_hbm)

    return gather_kernel(weight_pad, idx_row)


def kernel(inputs, weight):
    b, nc, h, w = inputs.shape
    x = jnp.transpose(inputs, (0, 2, 3, 1))
    flat = x.reshape(b * h * w, nc)
    sx = jnp.sum(flat ** 2, axis=1, keepdims=True)
    sw = jnp.sum(weight ** 2, axis=1)[None, :]
    idx3, part = _argmin_and_losssum(flat, (2.0 * weight).T, sx, sw)
    idx_row = idx3.reshape(1, flat.shape[0])
    weight_pad = jnp.pad(weight, ((0, 0), (0, 128 - _D)))
    quant_flat = _sc_gather(weight_pad, idx_row)[:, :_D]
    quant_flat = flat + (quant_flat - flat)
    total = jnp.sum(part[:, 0, 0])
    loss = (1.0 + _COMMIT) * total / (flat.shape[0] * nc)
    quantised = quant_flat.reshape(x.shape)
    return loss, jnp.transpose(quantised, (0, 3, 1, 2))


# speedup vs baseline: 4.1042x; 1.0588x over previous
import jax
import jax.numpy as jnp
from jax.experimental import pallas as pl
from jax.experimental.pallas import tpu as pltpu
from jax.experimental.pallas import tpu_sc as plsc

_N = 8192
_D = 32
_BM = 512
_CW = 256
_RG = 32
_GW = 128
_COMMIT = 0.25


def _dist_argmin_kernel(x_ref, wt2_ref, sx_ref, sw_ref, idx_ref, part_ref):
    x = x_ref[...]
    mm2 = jax.lax.dot_general(
        x, wt2_ref[...], (((1,), (0,)), ((), ())),
        preferred_element_type=jnp.float32,
        precision=jax.lax.Precision.DEFAULT)
    lane = jax.lax.broadcasted_iota(jnp.int32, (_RG, _CW), 1).astype(jnp.float32)
    lsum = jnp.float32(0.0)
    for r in range(_BM // _RG):
        rs = slice(r * _RG, (r + 1) * _RG)
        sx = sx_ref[rs, :]
        m = jnp.full((_RG, _CW), jnp.inf, jnp.float32)
        cidx = jnp.zeros((_RG, _CW), jnp.float32)
        for c in range(_N // _CW):
            cs = slice(c * _CW, (c + 1) * _CW)
            d = (sx + sw_ref[:, cs]) - mm2[rs, cs]
            better = d < m
            cidx = jnp.where(better, jnp.float32(c), cidx)
            m = jnp.minimum(m, d)
        bmin = jnp.min(m, axis=1, keepdims=True)
        jf = jnp.where(m == bmin, cidx * _CW + lane, jnp.float32(_N))
        idx_ref[0, rs] = jnp.min(jf, axis=1).astype(jnp.int32)
        lsum = lsum + jnp.sum(bmin)
    part_ref[0, 0, :] = jnp.broadcast_to(lsum, (128,))


def _argmin_and_losssum(flat, weight_t, sx, sw):
    grid = flat.shape[0] // _BM
    idx3, part = pl.pallas_call(
        _dist_argmin_kernel,
        grid=(grid,),
        in_specs=[
            pl.BlockSpec((_BM, _D), lambda i: (i, 0)),
            pl.BlockSpec((_D, _N), lambda i: (0, 0)),
            pl.BlockSpec((_BM, 1), lambda i: (i, 0)),
            pl.BlockSpec((1, _N), lambda i: (0, 0)),
        ],
        out_specs=[
            pl.BlockSpec((1, _BM), lambda i: (0, i)),
            pl.BlockSpec((1, 1, 128), lambda i: (i, 0, 0)),
        ],
        out_shape=[
            jax.ShapeDtypeStruct((1, flat.shape[0]), jnp.int32),
            jax.ShapeDtypeStruct((grid, 1, 128), jnp.float32),
        ],
    )(flat, weight_t, sx, sw)
    return idx3, part


def _sc_gather(weight_pad, idx_row):
    n_idx = idx_row.shape[1]
    lanes = weight_pad.shape[1]
    mesh = plsc.VectorSubcoreMesh(
        core_axis_name="core", subcore_axis_name="subcore")

    @pl.kernel(
        out_type=jax.ShapeDtypeStruct((n_idx, lanes), weight_pad.dtype),
        mesh=mesh)
    def gather_kernel(w_hbm, i_hbm, o_hbm):
        def body(i_vmem, o_vmem):
            pltpu.sync_copy(w_hbm.at[i_vmem.at[0]], o_vmem)

        pltpu.emit_pipeline(
            body,
            grid=(n_idx // _GW,),
            in_specs=[pl.BlockSpec((1, _GW), index_map=lambda i: (0, i))],
            out_specs=[pl.BlockSpec((_GW, lanes), index_map=lambda i: (i, 0))],
            core_axis_name="subcore",
            dimension_semantics=(pltpu.PARALLEL,),
        )(i_hbm, o_hbm)

    return gather_kernel(weight_pad, idx_row)


def kernel(inputs, weight):
    b, nc, h, w = inputs.shape
    x = jnp.transpose(inputs, (0, 2, 3, 1))
    flat = x.reshape(b * h * w, nc)
    sx = jnp.sum(flat ** 2, axis=1, keepdims=True)
    sw = jnp.sum(weight ** 2, axis=1)[None, :]
    idx_row, part = _argmin_and_losssum(flat, (2.0 * weight).T, sx, sw)
    weight_pad = jnp.pad(weight, ((0, 0), (0, 128 - _D)))
    quant_flat = _sc_gather(weight_pad, idx_row)[:, :_D]
    total = jnp.sum(part[:, 0, 0])
    loss = (1.0 + _COMMIT) * total / (flat.shape[0] * nc)
    quantised = quant_flat.reshape(x.shape)
    return loss, jnp.transpose(quantised, (0, 3, 1, 2))


# speedup vs baseline: 4.1828x; 1.0192x over previous
import jax
import jax.numpy as jnp
from jax.experimental import pallas as pl
from jax.experimental.pallas import tpu as pltpu
from jax.experimental.pallas import tpu_sc as plsc

_N = 8192
_D = 32
_BM = 512
_CW = 256
_RG = 32
_GW = 128
_COMMIT = 0.25


def _dist_argmin_kernel(x_ref, wt2_ref, sx_ref, sw_ref, idx_ref, part_ref):
    x = x_ref[...]
    mm2 = jax.lax.dot_general(
        x, wt2_ref[...], (((1,), (0,)), ((), ())),
        preferred_element_type=jnp.float32,
        precision=jax.lax.Precision.DEFAULT)
    lane = jax.lax.broadcasted_iota(jnp.int32, (_RG, _CW), 1).astype(jnp.float32)
    lsum = jnp.float32(0.0)
    for r in range(_BM // _RG):
        rs = slice(r * _RG, (r + 1) * _RG)
        sx = sx_ref[rs, :]
        m = jnp.full((_RG, _CW), jnp.inf, jnp.float32)
        cidx = jnp.zeros((_RG, _CW), jnp.float32)
        for c in range(_N // _CW):
            cs = slice(c * _CW, (c + 1) * _CW)
            d = (sx + sw_ref[:, cs]) - mm2[rs, cs]
            better = d < m
            cidx = jnp.where(better, jnp.float32(c), cidx)
            m = jnp.minimum(m, d)
        bmin = jnp.min(m, axis=1, keepdims=True)
        jf = jnp.where(m == bmin, cidx * _CW + lane, jnp.float32(_N))
        idx_ref[0, rs] = jnp.min(jf, axis=1).astype(jnp.int32)
        lsum = lsum + jnp.sum(bmin)
    part_ref[0, 0, :] = jnp.broadcast_to(lsum, (128,))


def _argmin_and_losssum(flat, weight_t, sx, sw, row0, nrows):
    grid = nrows // _BM
    blk0 = row0 // _BM
    idx3, part = pl.pallas_call(
        _dist_argmin_kernel,
        grid=(grid,),
        in_specs=[
            pl.BlockSpec((_BM, _D), lambda i: (blk0 + i, 0)),
            pl.BlockSpec((_D, _N), lambda i: (0, 0)),
            pl.BlockSpec((_BM, 1), lambda i: (blk0 + i, 0)),
            pl.BlockSpec((1, _N), lambda i: (0, 0)),
        ],
        out_specs=[
            pl.BlockSpec((1, _BM), lambda i: (0, i)),
            pl.BlockSpec((1, 1, 128), lambda i: (i, 0, 0)),
        ],
        out_shape=[
            jax.ShapeDtypeStruct((1, nrows), jnp.int32),
            jax.ShapeDtypeStruct((grid, 1, 128), jnp.float32),
        ],
    )(flat, weight_t, sx, sw)
    return idx3, part


def _sc_gather(weight_pad, idx_row):
    n_idx = idx_row.shape[1]
    lanes = weight_pad.shape[1]
    mesh = plsc.VectorSubcoreMesh(
        core_axis_name="core", subcore_axis_name="subcore")

    lanes = weight_pad.shape[1]

    @pl.kernel(
        out_type=jax.ShapeDtypeStruct((n_idx, lanes), weight_pad.dtype),
        mesh=mesh)
    def gather_kernel(w_hbm, i_hbm, o_hbm):
        def body(i_vmem, o_vmem):
            pltpu.sync_copy(w_hbm.at[i_vmem.at[0]], o_vmem)

        pltpu.emit_pipeline(
            body,
            grid=(n_idx // _GW,),
            in_specs=[pl.BlockSpec((1, _GW), index_map=lambda i: (0, i))],
            out_specs=[pl.BlockSpec((_GW, lanes), index_map=lambda i: (i, 0))],
            core_axis_name="subcore",
            dimension_semantics=(pltpu.PARALLEL,),
        )(i_hbm, o_hbm)

    return gather_kernel(weight_pad, idx_row)


def kernel(inputs, weight):
    b, nc, h, w = inputs.shape
    x = jnp.transpose(inputs, (0, 2, 3, 1))
    flat = x.reshape(b * h * w, nc)
    sx = jnp.sum(flat ** 2, axis=1, keepdims=True)
    sw = jnp.sum(weight ** 2, axis=1)[None, :]
    wt2 = (2.0 * weight).T
    weight_pad = jnp.pad(weight, ((0, 0), (0, 128 - _D)))
    nh = flat.shape[0] // 2
    q_halves, totals = [], []
    for hf in range(2):
        idx_row, part = _argmin_and_losssum(flat, wt2, sx, sw, hf * nh, nh)
        q_halves.append(_sc_gather(weight_pad, idx_row)[:, :_D])
        totals.append(jnp.sum(part[:, 0, 0]))
    quant_flat = jnp.concatenate(q_halves, axis=0)
    total = totals[0] + totals[1]
    loss = (1.0 + _COMMIT) * total / (flat.shape[0] * nc)
    quantised = quant_flat.reshape(x.shape)
    return loss, jnp.transpose(quantised, (0, 3, 1, 2))


# speedup vs baseline: 4.4186x; 1.0564x over previous
import jax
import jax.numpy as jnp
from jax.experimental import pallas as pl
from jax.experimental.pallas import tpu as pltpu
from jax.experimental.pallas import tpu_sc as plsc

_N = 8192
_D = 32
_BM = 1024
_CW = 128
_RG = 128
_GW = 256
_COMMIT = 0.25


def _dist_argmin_kernel(x_ref, wt2_ref, sx_ref, sw_ref, idx_ref, part_ref):
    lane = jax.lax.broadcasted_iota(jnp.int32, (_RG, _CW), 1).astype(jnp.float32)
    lsum = jnp.float32(0.0)
    for r in range(_BM // _RG):
        rs = slice(r * _RG, (r + 1) * _RG)
        xs = x_ref[rs, :]
        sx = sx_ref[rs, :]
        m = jnp.full((_RG, _CW), jnp.inf, jnp.float32)
        cidx = jnp.zeros((_RG, _CW), jnp.float32)
        for c in range(_N // _CW):
            cs = slice(c * _CW, (c + 1) * _CW)
            mm2 = jax.lax.dot_general(
                xs, wt2_ref[:, cs], (((1,), (0,)), ((), ())),
                preferred_element_type=jnp.float32,
                precision=jax.lax.Precision.DEFAULT)
            d = (sx + sw_ref[:, cs]) - mm2
            better = d < m
            cidx = jnp.where(better, jnp.float32(c), cidx)
            m = jnp.minimum(m, d)
        bmin = jnp.min(m, axis=1, keepdims=True)
        jf = jnp.where(m == bmin, cidx * _CW + lane, jnp.float32(_N))
        idx_ref[0, rs] = jnp.min(jf, axis=1).astype(jnp.int32)
        lsum = lsum + jnp.sum(bmin)
    part_ref[0, 0, :] = jnp.broadcast_to(lsum, (128,))


def _argmin_and_losssum(flat, weight_t, sx, sw, row0, nrows):
    grid = nrows // _BM
    blk0 = row0 // _BM
    idx3, part = pl.pallas_call(
        _dist_argmin_kernel,
        grid=(grid,),
        in_specs=[
            pl.BlockSpec((_BM, _D), lambda i: (blk0 + i, 0)),
            pl.BlockSpec((_D, _N), lambda i: (0, 0)),
            pl.BlockSpec((_BM, 1), lambda i: (blk0 + i, 0)),
            pl.BlockSpec((1, _N), lambda i: (0, 0)),
        ],
        out_specs=[
            pl.BlockSpec((1, _BM), lambda i: (0, i)),
            pl.BlockSpec((1, 1, 128), lambda i: (i, 0, 0)),
        ],
        out_shape=[
            jax.ShapeDtypeStruct((1, nrows), jnp.int32),
            jax.ShapeDtypeStruct((grid, 1, 128), jnp.float32),
        ],
    )(flat, weight_t, sx, sw)
    return idx3, part


def _sc_gather(weight_rows, idx_row):
    n_idx = idx_row.shape[1]
    lanes = weight_rows.shape[1]
    mesh = plsc.VectorSubcoreMesh(
        core_axis_name="core", subcore_axis_name="subcore")

    @pl.kernel(
        out_type=jax.ShapeDtypeStruct((n_idx, lanes), weight_rows.dtype),
        mesh=mesh)
    def gather_kernel(w_hbm, i_hbm, o_hbm):
        def body(i_vmem, o_vmem):
            pltpu.sync_copy(w_hbm.at[i_vmem.at[0]], o_vmem)

        pltpu.emit_pipeline(
            body,
            grid=(n_idx // _GW,),
            in_specs=[pl.BlockSpec((1, _GW), index_map=lambda i: (0, i))],
            out_specs=[pl.BlockSpec((_GW, lanes), index_map=lambda i: (i, 0))],
            core_axis_name="subcore",
            dimension_semantics=(pltpu.PARALLEL,),
        )(i_hbm, o_hbm)

    return gather_kernel(weight_rows, idx_row)


def kernel(inputs, weight):
    b, nc, h, w = inputs.shape
    x = jnp.transpose(inputs, (0, 2, 3, 1))
    flat = x.reshape(b * h * w, nc)
    sx = jnp.sum(flat ** 2, axis=1, keepdims=True)
    sw = jnp.sum(weight ** 2, axis=1)[None, :]
    wt2 = (2.0 * weight).T
    weight_pad = jnp.pad(weight, ((0, 0), (0, 128 - _D)))
    nh = flat.shape[0] // 2
    q_halves, totals = [], []
    for hf in range(2):
        idx_row, part = _argmin_and_losssum(flat, wt2, sx, sw, hf * nh, nh)
        q_halves.append(_sc_gather(weight_pad, idx_row)[:, :_D])
        totals.append(jnp.sum(part[:, 0, 0]))
    quant_flat = jnp.concatenate(q_halves, axis=0)
    total = totals[0] + totals[1]
    loss = (1.0 + _COMMIT) * total / (flat.shape[0] * nc)
    quantised = quant_flat.reshape(x.shape)
    return loss, jnp.transpose(quantised, (0, 3, 1, 2))
